# Initial kernel scaffold; baseline (speedup 1.0000x reference)
import jax
import jax.numpy as jnp
from jax.experimental import pallas as pl


def kernel(matrix, x_t):
    raise NotImplementedError("write your pallas kernel here")


# speedup vs baseline: 5.0538x; 5.0538x over previous
import functools

import jax
import jax.numpy as jnp
from jax import lax
from jax.experimental import pallas as pl
from jax.experimental.pallas import tpu as pltpu
from jax.experimental.pallas import tpu_sc as plsc

N_STATES = 16
M_SYMBOLS = 64
NC = 2
NS = 16
L = 16
NW = NC * NS

CHUNK = 2048
GROUPS = CHUNK // L

_LN2 = 0.6931471805599453
_SQRT2 = 1.4142135623730951


def _vlog(s):
    bits = lax.bitcast_convert_type(s, jnp.int32)
    e = (bits >> 23) - 127
    m = lax.bitcast_convert_type(
        (bits & 0x007FFFFF) | 0x3F800000, jnp.float32)
    big = m > _SQRT2
    m = jnp.where(big, m * 0.5, m)
    e = jnp.where(big, e + 1, e)
    z = (m - 1.0) / (m + 1.0)
    z2 = z * z
    poly = 1.0 + z2 * (1.0 / 3.0 + z2 * (0.2 + z2 * (1.0 / 7.0 + z2 / 9.0)))
    return e.astype(jnp.float32) * _LN2 + 2.0 * z * poly


def _shuffle(v, perm):
    return jnp.take_along_axis(v, perm, axis=0)


def _all_max(v, iota):
    for sh in (8, 4, 2, 1):
        v = jnp.maximum(v, _shuffle(v, iota ^ sh))
    return v


def _all_sum(v, iota):
    for sh in (8, 4, 2, 1):
        v = v + _shuffle(v, iota ^ sh)
    return v


def _body(mat_hbm, x_hbm, out_hbm, mat_v, table_v, idx_v, out_v):
    wid = lax.axis_index("s") * NC + lax.axis_index("c")
    b_per_w = x_hbm.shape[0] // NW
    nchunks = b_per_w // CHUNK

    iota = lax.iota(jnp.int32, L)

    pltpu.sync_copy(mat_hbm, mat_v)
    for r in range(N_STATES):
        a = [mat_v[r, pl.ds(k * L, L)] for k in range(M_SYMBOLS // L)]
        mxv = a[0]
        for k in range(1, len(a)):
            mxv = jnp.maximum(mxv, a[k])
        mx = _all_max(mxv, iota)
        ex = [jnp.exp(ak - mx) for ak in a]
        sv = ex[0]
        for k in range(1, len(ex)):
            sv = sv + ex[k]
        s = _all_sum(sv, iota)
        lse = mx + _vlog(s)
        for k in range(len(a)):
            plsc.store_scatter(
                table_v,
                [(iota + k * L) * N_STATES + r],
                a[k] - lse,
            )

    scat_rows0 = iota * N_STATES

    def group_body(g, _):
        v = idx_v[pl.ds(g * L, L)]
        v16 = v * N_STATES
        rowbase = scat_rows0 + g * (L * N_STATES)
        for n in range(N_STATES):
            vals = plsc.load_gather(table_v, [v16 + n])
            plsc.store_scatter(out_v, [rowbase + n], vals)
        return _

    for c in range(nchunks):
        base = wid * b_per_w + c * CHUNK
        pltpu.sync_copy(x_hbm.at[pl.ds(base, CHUNK)], idx_v)
        lax.fori_loop(0, GROUPS, group_body, 0, unroll=2)
        pltpu.sync_copy(out_v, out_hbm.at[pl.ds(base * N_STATES,
                                                CHUNK * N_STATES)])


@jax.jit
def _run(matrix, x_t):
    b = x_t.shape[0]
    mesh = plsc.VectorSubcoreMesh(
        core_axis_name="c", subcore_axis_name="s",
        num_cores=NC, num_subcores=NS)
    out_flat = pl.kernel(
        _body,
        out_type=jax.ShapeDtypeStruct((b * N_STATES,), jnp.float32),
        mesh=mesh,
        compiler_params=pltpu.CompilerParams(needs_layout_passes=False),
        scratch_types=[
            pltpu.VMEM((N_STATES, M_SYMBOLS), jnp.float32),
            pltpu.VMEM((M_SYMBOLS * N_STATES,), jnp.float32),
            pltpu.VMEM((CHUNK,), jnp.int32),
            pltpu.VMEM((CHUNK * N_STATES,), jnp.float32),
        ],
    )(matrix, x_t)
    return out_flat.reshape(b, N_STATES)


def kernel(matrix, x_t):
    return _run(matrix, x_t)


# speedup vs baseline: 5.2421x; 1.0373x over previous
import functools

import jax
import jax.numpy as jnp
from jax import lax
from jax.experimental import pallas as pl
from jax.experimental.pallas import tpu as pltpu
from jax.experimental.pallas import tpu_sc as plsc

N_STATES = 16
M_SYMBOLS = 64
NC = 2
NS = 16
L = 16
NW = NC * NS

CHUNK = 2048
GROUPS = CHUNK // L

_LN2 = 0.6931471805599453
_SQRT2 = 1.4142135623730951


def _vlog(s):
    bits = lax.bitcast_convert_type(s, jnp.int32)
    e = (bits >> 23) - 127
    m = lax.bitcast_convert_type(
        (bits & 0x007FFFFF) | 0x3F800000, jnp.float32)
    big = m > _SQRT2
    m = jnp.where(big, m * 0.5, m)
    e = jnp.where(big, e + 1, e)
    z = (m - 1.0) / (m + 1.0)
    z2 = z * z
    poly = 1.0 + z2 * (1.0 / 3.0 + z2 * (0.2 + z2 * (1.0 / 7.0 + z2 / 9.0)))
    return e.astype(jnp.float32) * _LN2 + 2.0 * z * poly


def _shuffle(v, perm):
    return jnp.take_along_axis(v, perm, axis=0)


def _all_max(v, iota):
    for sh in (8, 4, 2, 1):
        v = jnp.maximum(v, _shuffle(v, iota ^ sh))
    return v


def _all_sum(v, iota):
    for sh in (8, 4, 2, 1):
        v = v + _shuffle(v, iota ^ sh)
    return v


def _body(mat_hbm, x_hbm, out_hbm, mat_v, table_v, idx_v, out_v,
          in_sems, out_sems):
    wid = lax.axis_index("s") * NC + lax.axis_index("c")
    b_per_w = x_hbm.shape[0] // NW
    nchunks = b_per_w // CHUNK

    iota = lax.iota(jnp.int32, L)

    pltpu.sync_copy(mat_hbm, mat_v)
    for r in range(N_STATES):
        a = [mat_v[r, pl.ds(k * L, L)] for k in range(M_SYMBOLS // L)]
        mxv = a[0]
        for k in range(1, len(a)):
            mxv = jnp.maximum(mxv, a[k])
        mx = _all_max(mxv, iota)
        ex = [jnp.exp(ak - mx) for ak in a]
        sv = ex[0]
        for k in range(1, len(ex)):
            sv = sv + ex[k]
        s = _all_sum(sv, iota)
        lse = mx + _vlog(s)
        for k in range(len(a)):
            plsc.store_scatter(
                table_v,
                [(iota + k * L) * N_STATES + r],
                a[k] - lse,
            )

    scat_rows0 = iota * N_STATES

    OUTW = CHUNK * N_STATES

    def make_group_body(p):
        obase = scat_rows0 + p * OUTW

        def group_body(g, _):
            v = idx_v[pl.ds(p * CHUNK + g * L, L)]
            v16 = v * N_STATES
            rowbase = obase + g * (L * N_STATES)
            for n in range(N_STATES):
                vals = plsc.load_gather(table_v, [v16 + n])
                plsc.store_scatter(out_v, [rowbase + n], vals)
            return _
        return group_body

    def in_slice(c):
        return x_hbm.at[pl.ds(wid * b_per_w + c * CHUNK, CHUNK)]

    def out_slice(c):
        return out_hbm.at[pl.ds((wid * b_per_w + c * CHUNK) * N_STATES, OUTW)]

    def idx_buf(p):
        return idx_v.at[pl.ds(p * CHUNK, CHUNK)]

    def out_buf(p):
        return out_v.at[pl.ds(p * OUTW, OUTW)]

    pltpu.async_copy(in_slice(0), idx_buf(0), in_sems.at[0])
    for c in range(nchunks):
        p = c % 2
        if c + 1 < nchunks:
            pltpu.async_copy(in_slice(c + 1), idx_buf(1 - p),
                             in_sems.at[1 - p])
        pltpu.make_async_copy(in_slice(c), idx_buf(p), in_sems.at[p]).wait()
        if c >= 2:
            pltpu.make_async_copy(out_buf(p), out_slice(c - 2),
                                  out_sems.at[p]).wait()
        lax.fori_loop(0, GROUPS, make_group_body(p), 0, unroll=2)
        pltpu.async_copy(out_buf(p), out_slice(c), out_sems.at[p])
    for c in (nchunks - 2, nchunks - 1):
        p = c % 2
        pltpu.make_async_copy(out_buf(p), out_slice(c), out_sems.at[p]).wait()


@jax.jit
def _run(matrix, x_t):
    b = x_t.shape[0]
    mesh = plsc.VectorSubcoreMesh(
        core_axis_name="c", subcore_axis_name="s",
        num_cores=NC, num_subcores=NS)
    out_flat = pl.kernel(
        _body,
        out_type=jax.ShapeDtypeStruct((b * N_STATES,), jnp.float32),
        mesh=mesh,
        compiler_params=pltpu.CompilerParams(needs_layout_passes=False),
        scratch_types=[
            pltpu.VMEM((N_STATES, M_SYMBOLS), jnp.float32),
            pltpu.VMEM((M_SYMBOLS * N_STATES,), jnp.float32),
            pltpu.VMEM((2 * CHUNK,), jnp.int32),
            pltpu.VMEM((2 * CHUNK * N_STATES,), jnp.float32),
            pltpu.SemaphoreType.DMA((2,)),
            pltpu.SemaphoreType.DMA((2,)),
        ],
    )(matrix, x_t)
    return out_flat.reshape(b, N_STATES)


def kernel(matrix, x_t):
    return _run(matrix, x_t)


# speedup vs baseline: 6.1021x; 1.1641x over previous
import functools

import jax
import jax.numpy as jnp
from jax import lax
from jax.experimental import pallas as pl
from jax.experimental.pallas import tpu as pltpu
from jax.experimental.pallas import tpu_sc as plsc

N_STATES = 16
M_SYMBOLS = 64
NC = 2
NS = 16
L = 16
NW = NC * NS

CHUNK = 2048
GROUPS = CHUNK // L

_LN2 = 0.6931471805599453
_SQRT2 = 1.4142135623730951


def _vlog(s):
    bits = lax.bitcast_convert_type(s, jnp.int32)
    e = (bits >> 23) - 127
    m = lax.bitcast_convert_type(
        (bits & 0x007FFFFF) | 0x3F800000, jnp.float32)
    big = m > _SQRT2
    m = jnp.where(big, m * 0.5, m)
    e = jnp.where(big, e + 1, e)
    z = (m - 1.0) / (m + 1.0)
    z2 = z * z
    poly = 1.0 + z2 * (1.0 / 3.0 + z2 * (0.2 + z2 * (1.0 / 7.0 + z2 / 9.0)))
    return e.astype(jnp.float32) * _LN2 + 2.0 * z * poly


def _shuffle(v, perm):
    return jnp.take_along_axis(v, perm, axis=0)


def _all_max(v, iota):
    for sh in (8, 4, 2, 1):
        v = jnp.maximum(v, _shuffle(v, iota ^ sh))
    return v


def _all_sum(v, iota):
    for sh in (8, 4, 2, 1):
        v = v + _shuffle(v, iota ^ sh)
    return v


def _body(mat_hbm, x_hbm, out_hbm, mat_v, table_v, idx_v, out_v,
          in_sems, out_sems):
    wid = lax.axis_index("s") * NC + lax.axis_index("c")
    b_per_w = x_hbm.shape[0] // NW
    nchunks = b_per_w // CHUNK

    iota = lax.iota(jnp.int32, L)

    pltpu.sync_copy(mat_hbm, mat_v)
    for r in range(N_STATES):
        a = [mat_v[r, pl.ds(k * L, L)] for k in range(M_SYMBOLS // L)]
        mxv = a[0]
        for k in range(1, len(a)):
            mxv = jnp.maximum(mxv, a[k])
        mx = _all_max(mxv, iota)
        ex = [jnp.exp(ak - mx) for ak in a]
        sv = ex[0]
        for k in range(1, len(ex)):
            sv = sv + ex[k]
        s = _all_sum(sv, iota)
        lse = mx + _vlog(s)
        for k in range(len(a)):
            plsc.store_scatter(
                table_v,
                [(iota + k * L) * N_STATES + r],
                a[k] - lse,
            )

    scat_rows0 = iota * N_STATES

    OUTW = CHUNK * N_STATES

    def run_groups(p):
        obase = scat_rows0 + p * OUTW

        def group_body(g):
            v = idx_v[pl.ds(p * CHUNK + g * L, L)]
            v16 = v * N_STATES
            rowbase = obase + g * (L * N_STATES)
            for n in range(N_STATES):
                vals = plsc.load_gather(table_v, [v16 + n])
                plsc.store_scatter(out_v, [rowbase + n], vals)

        plsc.parallel_loop(0, GROUPS, step=1, unroll=2)(group_body)

    def in_slice(c):
        return x_hbm.at[pl.ds(wid * b_per_w + c * CHUNK, CHUNK)]

    def out_slice(c):
        return out_hbm.at[pl.ds((wid * b_per_w + c * CHUNK) * N_STATES, OUTW)]

    def idx_buf(p):
        return idx_v.at[pl.ds(p * CHUNK, CHUNK)]

    def out_buf(p):
        return out_v.at[pl.ds(p * OUTW, OUTW)]

    pltpu.async_copy(in_slice(0), idx_buf(0), in_sems.at[0])
    for c in range(nchunks):
        p = c % 2
        if c + 1 < nchunks:
            pltpu.async_copy(in_slice(c + 1), idx_buf(1 - p),
                             in_sems.at[1 - p])
        pltpu.make_async_copy(in_slice(c), idx_buf(p), in_sems.at[p]).wait()
        if c >= 2:
            pltpu.make_async_copy(out_buf(p), out_slice(c - 2),
                                  out_sems.at[p]).wait()
        run_groups(p)
        pltpu.async_copy(out_buf(p), out_slice(c), out_sems.at[p])
    for c in (nchunks - 2, nchunks - 1):
        p = c % 2
        pltpu.make_async_copy(out_buf(p), out_slice(c), out_sems.at[p]).wait()


@jax.jit
def _run(matrix, x_t):
    b = x_t.shape[0]
    mesh = plsc.VectorSubcoreMesh(
        core_axis_name="c", subcore_axis_name="s",
        num_cores=NC, num_subcores=NS)
    out_flat = pl.kernel(
        _body,
        out_type=jax.ShapeDtypeStruct((b * N_STATES,), jnp.float32),
        mesh=mesh,
        compiler_params=pltpu.CompilerParams(needs_layout_passes=False),
        scratch_types=[
            pltpu.VMEM((N_STATES, M_SYMBOLS), jnp.float32),
            pltpu.VMEM((M_SYMBOLS * N_STATES,), jnp.float32),
            pltpu.VMEM((2 * CHUNK,), jnp.int32),
            pltpu.VMEM((2 * CHUNK * N_STATES,), jnp.float32),
            pltpu.SemaphoreType.DMA((2,)),
            pltpu.SemaphoreType.DMA((2,)),
        ],
    )(matrix, x_t)
    return out_flat.reshape(b, N_STATES)


def kernel(matrix, x_t):
    return _run(matrix, x_t)
